# baseline (speedup 1.0000x reference)
import jax
import jax.numpy as jnp
from jax import lax
from jax.experimental import pallas as pl
from jax.experimental.pallas import tpu as pltpu
from jax.experimental.pallas import tpu_sc as plsc

NC, NS = 2, 16
NW = NC * NS
LANES = 16

TOTAL, D, B = 32768, 512, 16
TC_ROWS = 20480
SC_ROWS = TOTAL - TC_ROWS
ROWS_W = SC_ROWS // NW
BLK = 32
NRING = 4
NBLK = ROWS_W // BLK
CG = D // LANES
TBLK = 4096
SUB = 512
NSUB = TBLK // SUB


def _sc_body(vals, ps_hbm, part_hbm,
             ps_v, bnd_s, buf0, buf1, buf2, buf3, part_v,
             sem0, sem1, sem2, sem3):
    cid = lax.axis_index("c")
    sid = lax.axis_index("s")
    wid = sid * NC + cid
    r0 = TC_ROWS + wid * ROWS_W
    bufs = (buf0, buf1, buf2, buf3)
    sems = (sem0, sem1, sem2, sem3)

    pltpu.sync_copy(ps_hbm, ps_v)
    v0 = ps_v[...]
    for j in range(LANES):
        bnd_s[j] = v0[j]
    bnd_s[B] = TOTAL

    minus_inf = jnp.full((LANES,), -jnp.inf, jnp.float32)

    def init_body(b, carry):
        for c in range(CG):
            part_v[b, pl.ds(c * LANES, LANES)] = minus_inf
        return carry

    lax.fori_loop(0, B, init_body, 0)

    def copy(k, slot):
        return pltpu.make_async_copy(
            vals.at[pl.ds(r0 + k * BLK, BLK)], bufs[slot], sems[slot])

    for k0 in range(NRING - 1):
        copy(k0, k0).start()

    def blk_body(j, carry):
        for phase in range(NRING):
            k = NRING * j + phase
            slot = phase
            nslot = (phase + NRING - 1) % NRING
            copy(k, slot).wait()

            @pl.when(k + NRING - 1 < NBLK)
            def _():
                copy(k + NRING - 1, nslot).start()

            buf = bufs[slot]
            gb0 = r0 + k * BLK

            def seg_body(b, carry2):
                l = jnp.maximum(bnd_s[b] - gb0, 0)
                h = jnp.minimum(bnd_s[b + 1] - gb0, BLK)

                @pl.when(l < h)
                def _():
                    def rbody(r, accs):
                        return tuple(
                            jnp.maximum(a, buf[r, pl.ds(c * LANES, LANES)])
                            for c, a in enumerate(accs))
                    init = tuple(part_v[b, pl.ds(c * LANES, LANES)]
                                 for c in range(CG))
                    accs = lax.fori_loop(l, h, rbody, init)
                    for c in range(CG):
                        part_v[b, pl.ds(c * LANES, LANES)] = accs[c]
                return carry2

            lax.fori_loop(0, B, seg_body, 0)
        return carry

    lax.fori_loop(0, NBLK // NRING, blk_body, 0)
    pltpu.sync_copy(part_v, part_hbm.at[wid])


_sc_partial_max = pl.kernel(
    _sc_body,
    out_type=jax.ShapeDtypeStruct((NW, B, D), jnp.float32),
    mesh=plsc.VectorSubcoreMesh(core_axis_name="c", subcore_axis_name="s"),
    scratch_types=[
        pltpu.VMEM((LANES,), jnp.int32),
        pltpu.SMEM((B + 1,), jnp.int32),
        pltpu.VMEM((BLK, D), jnp.float32),
        pltpu.VMEM((BLK, D), jnp.float32),
        pltpu.VMEM((BLK, D), jnp.float32),
        pltpu.VMEM((BLK, D), jnp.float32),
        pltpu.VMEM((B, D), jnp.float32),
        pltpu.SemaphoreType.DMA,
        pltpu.SemaphoreType.DMA,
        pltpu.SemaphoreType.DMA,
        pltpu.SemaphoreType.DMA,
    ],
)


def _tc_body(ps_ref, x_ref, o_ref):
    i = pl.program_id(0)

    @pl.when(i == 0)
    def _():
        o_ref[...] = jnp.full((B, D), -jnp.inf, jnp.float32)

    for s in range(NSUB):
        base = i * TBLK + s * SUB
        m = jnp.max(x_ref[pl.ds(s * SUB, SUB), :], axis=0, keepdims=True)

        jb = jnp.int32(0)
        nb = jnp.int32(0)
        for b in range(1, B):
            p = ps_ref[b]
            jb = jnp.where(p <= base, jnp.int32(b), jb)
            nb = nb + jnp.where((p > base) & (p < base + SUB), 1, 0)

        @pl.when(nb == 0)
        def _(jb=jb, m=m):
            o_ref[pl.ds(jb, 1), :] = jnp.maximum(o_ref[pl.ds(jb, 1), :], m)

        @pl.when(nb > 0)
        def _(base=base, s=s):
            rows = lax.broadcasted_iota(jnp.int32, (SUB, 1), 0)

            def fb(b, carry):
                l = jnp.clip(ps_ref[b] - base, 0, SUB)
                h = jnp.clip(ps_ref[b + 1] - base, 0, SUB)

                @pl.when(l < h)
                def _():
                    xs = x_ref[pl.ds(s * SUB, SUB), :]
                    mask = (rows >= l) & (rows < h)
                    mm = jnp.max(jnp.where(mask, xs, -jnp.inf), axis=0,
                                 keepdims=True)
                    o_ref[pl.ds(b, 1), :] = jnp.maximum(
                        o_ref[pl.ds(b, 1), :], mm)
                return carry

            lax.fori_loop(0, B, fb, 0)


_tc_partial_max = pl.pallas_call(
    _tc_body,
    grid_spec=pltpu.PrefetchScalarGridSpec(
        num_scalar_prefetch=1,
        grid=(TC_ROWS // TBLK,),
        in_specs=[pl.BlockSpec((TBLK, D), lambda i, ps: (i, 0))],
        out_specs=pl.BlockSpec((B, D), lambda i, ps: (0, 0)),
    ),
    out_shape=jax.ShapeDtypeStruct((B, D), jnp.float32),
)


def _merge_body(psc_ref, ptc_ref, o_ref):
    o_ref[...] = jnp.maximum(jnp.max(psc_ref[...], axis=0), ptc_ref[...])


_merge = pl.pallas_call(
    _merge_body,
    out_shape=jax.ShapeDtypeStruct((B, D), jnp.float32),
)


def kernel(values, prefix_sum):
    ps16 = lax.slice(prefix_sum, (0,), (LANES,))
    partials_sc = _sc_partial_max(values, ps16)
    part_tc = _tc_partial_max(prefix_sum, values)
    return _merge(partials_sc, part_tc)


# speedup vs baseline: 1.0249x; 1.0249x over previous
import jax
import jax.numpy as jnp
from jax import lax
from jax.experimental import pallas as pl
from jax.experimental.pallas import tpu as pltpu
from jax.experimental.pallas import tpu_sc as plsc

NC, NS = 1, 16
NW = NC * NS
LANES = 16

TOTAL, D, B = 32768, 512, 16
TC_ROWS = 28672
SC_ROWS = TOTAL - TC_ROWS
ROWS_W = SC_ROWS // NW
BLK = 32
NRING = 4
NBLK = ROWS_W // BLK
CG = D // LANES
TBLK = 2048
SUB = 512
NSUB = TBLK // SUB


def _sc_body(vals, ps_hbm, part_hbm,
             ps_v, bnd_s, buf0, buf1, buf2, buf3, part_v,
             sem0, sem1, sem2, sem3):
    cid = lax.axis_index("c")
    sid = lax.axis_index("s")
    wid = sid * NC + cid
    r0 = TC_ROWS + wid * ROWS_W
    bufs = (buf0, buf1, buf2, buf3)
    sems = (sem0, sem1, sem2, sem3)

    pltpu.sync_copy(ps_hbm, ps_v)
    v0 = ps_v[...]
    for j in range(LANES):
        bnd_s[j] = v0[j]
    bnd_s[B] = TOTAL

    minus_inf = jnp.full((LANES,), -jnp.inf, jnp.float32)

    def init_body(b, carry):
        for c in range(CG):
            part_v[b, pl.ds(c * LANES, LANES)] = minus_inf
        return carry

    lax.fori_loop(0, B, init_body, 0)

    def copy(k, slot):
        return pltpu.make_async_copy(
            vals.at[pl.ds(r0 + k * BLK, BLK)], bufs[slot], sems[slot])

    for k0 in range(NRING - 1):
        copy(k0, k0).start()

    def blk_body(j, carry):
        for phase in range(NRING):
            k = NRING * j + phase
            slot = phase
            nslot = (phase + NRING - 1) % NRING
            copy(k, slot).wait()

            @pl.when(k + NRING - 1 < NBLK)
            def _():
                copy(k + NRING - 1, nslot).start()

            buf = bufs[slot]
            gb0 = r0 + k * BLK

            def seg_body(b, carry2):
                l = jnp.maximum(bnd_s[b] - gb0, 0)
                h = jnp.minimum(bnd_s[b + 1] - gb0, BLK)

                @pl.when(l < h)
                def _():
                    def rbody(r, accs):
                        return tuple(
                            jnp.maximum(a, buf[r, pl.ds(c * LANES, LANES)])
                            for c, a in enumerate(accs))
                    init = tuple(part_v[b, pl.ds(c * LANES, LANES)]
                                 for c in range(CG))
                    accs = lax.fori_loop(l, h, rbody, init)
                    for c in range(CG):
                        part_v[b, pl.ds(c * LANES, LANES)] = accs[c]
                return carry2

            lax.fori_loop(0, B, seg_body, 0)
        return carry

    lax.fori_loop(0, NBLK // NRING, blk_body, 0)
    pltpu.sync_copy(part_v, part_hbm.at[wid])


_sc_partial_max = pl.kernel(
    _sc_body,
    out_type=jax.ShapeDtypeStruct((NW, B, D), jnp.float32),
    mesh=plsc.VectorSubcoreMesh(core_axis_name="c", subcore_axis_name="s",
                                num_cores=NC),
    scratch_types=[
        pltpu.VMEM((LANES,), jnp.int32),
        pltpu.SMEM((B + 1,), jnp.int32),
        pltpu.VMEM((BLK, D), jnp.float32),
        pltpu.VMEM((BLK, D), jnp.float32),
        pltpu.VMEM((BLK, D), jnp.float32),
        pltpu.VMEM((BLK, D), jnp.float32),
        pltpu.VMEM((B, D), jnp.float32),
        pltpu.SemaphoreType.DMA,
        pltpu.SemaphoreType.DMA,
        pltpu.SemaphoreType.DMA,
        pltpu.SemaphoreType.DMA,
    ],
)


def _tc_body(ps_ref, x_ref, o_ref):
    i = pl.program_id(0)

    @pl.when(i == 0)
    def _():
        o_ref[...] = jnp.full((B, D), -jnp.inf, jnp.float32)

    for s in range(NSUB):
        base = i * TBLK + s * SUB
        m = jnp.max(x_ref[pl.ds(s * SUB, SUB), :], axis=0, keepdims=True)

        jb = jnp.int32(0)
        nb = jnp.int32(0)
        for b in range(1, B):
            p = ps_ref[b]
            jb = jnp.where(p <= base, jnp.int32(b), jb)
            nb = nb + jnp.where((p > base) & (p < base + SUB), 1, 0)

        @pl.when(nb == 0)
        def _(jb=jb, m=m):
            o_ref[pl.ds(jb, 1), :] = jnp.maximum(o_ref[pl.ds(jb, 1), :], m)

        @pl.when(nb > 0)
        def _(base=base, s=s):
            rows = lax.broadcasted_iota(jnp.int32, (SUB, 1), 0)

            def fb(b, carry):
                l = jnp.clip(ps_ref[b] - base, 0, SUB)
                h = jnp.clip(ps_ref[b + 1] - base, 0, SUB)

                @pl.when(l < h)
                def _():
                    xs = x_ref[pl.ds(s * SUB, SUB), :]
                    mask = (rows >= l) & (rows < h)
                    mm = jnp.max(jnp.where(mask, xs, -jnp.inf), axis=0,
                                 keepdims=True)
                    o_ref[pl.ds(b, 1), :] = jnp.maximum(
                        o_ref[pl.ds(b, 1), :], mm)
                return carry

            lax.fori_loop(0, B, fb, 0)


_tc_partial_max = pl.pallas_call(
    _tc_body,
    grid_spec=pltpu.PrefetchScalarGridSpec(
        num_scalar_prefetch=1,
        grid=(TC_ROWS // TBLK,),
        in_specs=[pl.BlockSpec((TBLK, D), lambda i, ps: (i, 0))],
        out_specs=pl.BlockSpec((B, D), lambda i, ps: (0, 0)),
    ),
    out_shape=jax.ShapeDtypeStruct((B, D), jnp.float32),
)


def _merge_body(psc_ref, ptc_ref, o_ref):
    o_ref[...] = jnp.maximum(jnp.max(psc_ref[...], axis=0), ptc_ref[...])


_merge = pl.pallas_call(
    _merge_body,
    out_shape=jax.ShapeDtypeStruct((B, D), jnp.float32),
)


def kernel(values, prefix_sum):
    ps16 = lax.slice(prefix_sum, (0,), (LANES,))
    partials_sc = _sc_partial_max(values, ps16)
    part_tc = _tc_partial_max(prefix_sum, values)
    return _merge(partials_sc, part_tc)


# speedup vs baseline: 1.0619x; 1.0361x over previous
import jax
import jax.numpy as jnp
from jax import lax
from jax.experimental import pallas as pl
from jax.experimental.pallas import tpu as pltpu
from jax.experimental.pallas import tpu_sc as plsc

NC, NS = 1, 16
NW = NC * NS
LANES = 16

TOTAL, D, B = 32768, 512, 16
TC_ROWS = 26624
SC_ROWS = TOTAL - TC_ROWS
ROWS_W = SC_ROWS // NW
BLK = 32
NRING = 4
NBLK = ROWS_W // BLK
CG = D // LANES
TBLK = 2048
SUB = 512
NSUB = TBLK // SUB


def _sc_body(vals, ps_hbm, part_hbm,
             ps_v, bnd_s, buf0, buf1, buf2, buf3, part_v,
             sem0, sem1, sem2, sem3):
    cid = lax.axis_index("c")
    sid = lax.axis_index("s")
    wid = sid * NC + cid
    r0 = TC_ROWS + wid * ROWS_W
    bufs = (buf0, buf1, buf2, buf3)
    sems = (sem0, sem1, sem2, sem3)

    pltpu.sync_copy(ps_hbm, ps_v)
    v0 = ps_v[...]
    for j in range(LANES):
        bnd_s[j] = v0[j]
    bnd_s[B] = TOTAL

    minus_inf = jnp.full((LANES,), -jnp.inf, jnp.float32)

    def init_body(b, carry):
        for c in range(CG):
            part_v[b, pl.ds(c * LANES, LANES)] = minus_inf
        return carry

    lax.fori_loop(0, B, init_body, 0)

    def copy(k, slot):
        return pltpu.make_async_copy(
            vals.at[pl.ds(r0 + k * BLK, BLK)], bufs[slot], sems[slot])

    for k0 in range(NRING - 1):
        copy(k0, k0).start()

    def blk_body(j, carry):
        for phase in range(NRING):
            k = NRING * j + phase
            slot = phase
            nslot = (phase + NRING - 1) % NRING
            copy(k, slot).wait()

            @pl.when(k + NRING - 1 < NBLK)
            def _():
                copy(k + NRING - 1, nslot).start()

            buf = bufs[slot]
            gb0 = r0 + k * BLK

            def seg_body(b, carry2):
                l = jnp.maximum(bnd_s[b] - gb0, 0)
                h = jnp.minimum(bnd_s[b + 1] - gb0, BLK)

                @pl.when(l < h)
                def _():
                    def rbody(r, accs):
                        return tuple(
                            jnp.maximum(a, buf[r, pl.ds(c * LANES, LANES)])
                            for c, a in enumerate(accs))
                    init = tuple(part_v[b, pl.ds(c * LANES, LANES)]
                                 for c in range(CG))
                    accs = lax.fori_loop(l, h, rbody, init)
                    for c in range(CG):
                        part_v[b, pl.ds(c * LANES, LANES)] = accs[c]
                return carry2

            lax.fori_loop(0, B, seg_body, 0)
        return carry

    lax.fori_loop(0, NBLK // NRING, blk_body, 0)
    pltpu.sync_copy(part_v, part_hbm.at[wid])


_sc_partial_max = pl.kernel(
    _sc_body,
    out_type=jax.ShapeDtypeStruct((NW, B, D), jnp.float32),
    mesh=plsc.VectorSubcoreMesh(core_axis_name="c", subcore_axis_name="s",
                                num_cores=NC),
    scratch_types=[
        pltpu.VMEM((LANES,), jnp.int32),
        pltpu.SMEM((B + 1,), jnp.int32),
        pltpu.VMEM((BLK, D), jnp.float32),
        pltpu.VMEM((BLK, D), jnp.float32),
        pltpu.VMEM((BLK, D), jnp.float32),
        pltpu.VMEM((BLK, D), jnp.float32),
        pltpu.VMEM((B, D), jnp.float32),
        pltpu.SemaphoreType.DMA,
        pltpu.SemaphoreType.DMA,
        pltpu.SemaphoreType.DMA,
        pltpu.SemaphoreType.DMA,
    ],
)


def _tc_body(ps_ref, x_ref, o_ref):
    i = pl.program_id(0)

    @pl.when(i == 0)
    def _():
        o_ref[...] = jnp.full((B, D), -jnp.inf, jnp.float32)

    for s in range(NSUB):
        base = i * TBLK + s * SUB
        m = jnp.max(x_ref[pl.ds(s * SUB, SUB), :], axis=0, keepdims=True)

        jb = jnp.int32(0)
        nb = jnp.int32(0)
        for b in range(1, B):
            p = ps_ref[b]
            jb = jnp.where(p <= base, jnp.int32(b), jb)
            nb = nb + jnp.where((p > base) & (p < base + SUB), 1, 0)

        @pl.when(nb == 0)
        def _(jb=jb, m=m):
            o_ref[pl.ds(jb, 1), :] = jnp.maximum(o_ref[pl.ds(jb, 1), :], m)

        @pl.when(nb > 0)
        def _(base=base, s=s):
            rows = lax.broadcasted_iota(jnp.int32, (SUB, 1), 0)

            def fb(b, carry):
                l = jnp.clip(ps_ref[b] - base, 0, SUB)
                h = jnp.clip(ps_ref[b + 1] - base, 0, SUB)

                @pl.when(l < h)
                def _():
                    xs = x_ref[pl.ds(s * SUB, SUB), :]
                    mask = (rows >= l) & (rows < h)
                    mm = jnp.max(jnp.where(mask, xs, -jnp.inf), axis=0,
                                 keepdims=True)
                    o_ref[pl.ds(b, 1), :] = jnp.maximum(
                        o_ref[pl.ds(b, 1), :], mm)
                return carry

            lax.fori_loop(0, B, fb, 0)


_tc_partial_max = pl.pallas_call(
    _tc_body,
    grid_spec=pltpu.PrefetchScalarGridSpec(
        num_scalar_prefetch=1,
        grid=(TC_ROWS // TBLK,),
        in_specs=[pl.BlockSpec((TBLK, D), lambda i, ps: (i, 0))],
        out_specs=pl.BlockSpec((B, D), lambda i, ps: (0, 0)),
    ),
    out_shape=jax.ShapeDtypeStruct((B, D), jnp.float32),
)


def _merge_body(psc_ref, ptc_ref, o_ref):
    o_ref[...] = jnp.maximum(jnp.max(psc_ref[...], axis=0), ptc_ref[...])


_merge = pl.pallas_call(
    _merge_body,
    out_shape=jax.ShapeDtypeStruct((B, D), jnp.float32),
)


def kernel(values, prefix_sum):
    ps16 = lax.slice(prefix_sum, (0,), (LANES,))
    partials_sc = _sc_partial_max(values, ps16)
    part_tc = _tc_partial_max(prefix_sum, values)
    return _merge(partials_sc, part_tc)
